# baseline (speedup 1.0000x reference)
import jax
import jax.numpy as jnp
from jax import lax
from jax.experimental import pallas as pl
from jax.experimental.pallas import tpu as pltpu
from jax.experimental.pallas import tpu_sc as plsc

RADIUS = 0.2
NSAMPLE = 32

B = 2
N = 8192
S = 1024
C = 64

NUM_TILES = 16
CPT = S // NUM_TILES
LANES = 16
KCHUNK = 8
SSTEP = KCHUNK * LANES
NSUPER = N // SSTEP
CH_PER_TILE = C // NUM_TILES
SCHUNK = 128
NSCHUNK = S // SCHUNK
BUFSZ = NSAMPLE + SSTEP


def _body(xyz_hbm, cen_hbm, feat_hbm, out_hbm,
          pts_v, cen_v, buf_v, idxstage_v, idx_sh, idx_v, feat_v, stage_v,
          feat_sem, idx_sem):
    b = lax.axis_index("c")
    t = lax.axis_index("s")
    r2 = RADIUS * RADIUS

    feat_copies = []
    for q in range(CH_PER_TILE):
        ch = t * CH_PER_TILE + q
        feat_copies.append(pltpu.async_copy(
            feat_hbm.at[b, ch], feat_v.at[pl.ds(q * N, N)], feat_sem))

    pltpu.sync_copy(xyz_hbm.at[b], pts_v)
    pltpu.sync_copy(cen_hbm.at[b], cen_v)

    lane = lax.iota(jnp.int32, LANES)
    one = jnp.ones((LANES,), jnp.int32)

    def center_body(ci, _):
        s = t * CPT + ci
        sv = jnp.full((LANES,), s, jnp.int32)
        cx = plsc.load_gather(cen_v, [sv])
        cy = plsc.load_gather(cen_v, [sv + S])
        cz = plsc.load_gather(cen_v, [sv + 2 * S])

        def dist_mask(base):
            xs = pts_v[pl.ds(base, LANES)]
            ys = pts_v[pl.ds(base + N, LANES)]
            zs = pts_v[pl.ds(base + 2 * N, LANES)]
            dx = xs - cx
            dy = ys - cy
            dz = zs - cz
            return dx * dx + dy * dy + dz * dz <= r2

        def cond(carry):
            i, pref = carry
            return jnp.logical_and(i < NSUPER, pref[0] < NSAMPLE)

        def body(carry):
            i, pref = carry
            base = pl.multiple_of(i * SSTEP, SSTEP)
            for c in range(KCHUNK):
                m = dist_mask(base + c * LANES)
                pos = pref + plsc.cumsum(jnp.where(m, one, 0)) - 1
                plsc.store_scatter(buf_v, [pos],
                                   lane + (base + c * LANES), mask=m)
                pref = pref + plsc.all_reduce_population_count(m)
            return i + 1, pref

        _, pref = lax.while_loop(
            cond, body, (jnp.int32(0), jnp.zeros((LANES,), jnp.int32)))
        count = pref[0]

        first = plsc.load_gather(buf_v, [jnp.zeros((LANES,), jnp.int32)])
        fill = jnp.where(
            jnp.full((LANES,), count) == 0,
            jnp.full((LANES,), N - 1, jnp.int32), first)
        for j in range(NSAMPLE // LANES):
            pos = lane + j * LANES
            cur = buf_v[pl.ds(j * LANES, LANES)]
            res = jnp.where(pos < jnp.full((LANES,), count), cur, fill)
            idxstage_v[pl.ds(ci * NSAMPLE + j * LANES, LANES)] = res
        return 0

    lax.fori_loop(0, CPT, center_body, 0)

    pltpu.sync_copy(idxstage_v, idx_sh.at[pl.ds(t * CPT * NSAMPLE,
                                                CPT * NSAMPLE)])
    plsc.subcore_barrier()
    idx_copy = pltpu.async_copy(idx_sh, idx_v, idx_sem)

    def xyz_body(ci, _):
        s = t * CPT + ci
        sv = jnp.full((LANES,), s, jnp.int32)
        cens = [plsc.load_gather(cen_v, [sv + d * S]) for d in range(3)]
        for j in range(NSAMPLE // LANES):
            idxv = idxstage_v[pl.ds(ci * NSAMPLE + j * LANES, LANES)]
            for d in range(3):
                vals = plsc.load_gather(pts_v, [idxv + d * N]) - cens[d]
                stage_v[pl.ds(d * CPT * NSAMPLE + ci * NSAMPLE + j * LANES,
                              LANES)] = vals
        return 0

    lax.fori_loop(0, CPT, xyz_body, 0)
    for d in range(3):
        pltpu.sync_copy(
            stage_v.at[pl.ds(d * CPT * NSAMPLE, CPT * NSAMPLE)],
            out_hbm.at[b, d, pl.ds(t * CPT * NSAMPLE, CPT * NSAMPLE)])

    idx_copy.wait()
    for cp in feat_copies:
        cp.wait()

    def chunk_body(k, _):
        def cbody(ci, _):
            s = k * SCHUNK + ci
            for j in range(NSAMPLE // LANES):
                idxv = idx_v[pl.ds(s * NSAMPLE + j * LANES, LANES)]
                for q in range(CH_PER_TILE):
                    vals = plsc.load_gather(feat_v, [idxv + q * N])
                    stage_v[pl.ds(q * SCHUNK * NSAMPLE + ci * NSAMPLE
                                  + j * LANES, LANES)] = vals
            return 0
        lax.fori_loop(0, SCHUNK, cbody, 0)
        for q in range(CH_PER_TILE):
            ch = t * CH_PER_TILE + q
            pltpu.sync_copy(
                stage_v.at[pl.ds(q * SCHUNK * NSAMPLE, SCHUNK * NSAMPLE)],
                out_hbm.at[b, 3 + ch, pl.ds(k * SCHUNK * NSAMPLE,
                                            SCHUNK * NSAMPLE)])
        return 0

    lax.fori_loop(0, NSCHUNK, chunk_body, 0)


@jax.jit
def kernel(xyz, center_xyz, features):
    xyz_t = jnp.transpose(xyz, (0, 2, 1)).reshape(B, 3 * N)
    cen_t = jnp.transpose(center_xyz, (0, 2, 1)).reshape(B, 3 * S)

    mesh = plsc.VectorSubcoreMesh(core_axis_name="c", subcore_axis_name="s",
                                  num_cores=2, num_subcores=NUM_TILES)
    run = pl.kernel(
        _body,
        out_type=jax.ShapeDtypeStruct((B, 3 + C, S * NSAMPLE), jnp.float32),
        mesh=mesh,
        compiler_params=pltpu.CompilerParams(needs_layout_passes=False),
        scratch_types=[
            pltpu.VMEM((3 * N,), jnp.float32),
            pltpu.VMEM((3 * S,), jnp.float32),
            pltpu.VMEM((BUFSZ,), jnp.int32),
            pltpu.VMEM((CPT * NSAMPLE,), jnp.int32),
            pltpu.VMEM_SHARED((S * NSAMPLE,), jnp.int32),
            pltpu.VMEM((S * NSAMPLE,), jnp.int32),
            pltpu.VMEM((CH_PER_TILE * N,), jnp.float32),
            pltpu.VMEM((CH_PER_TILE * SCHUNK * NSAMPLE,),
                       jnp.float32),
            pltpu.SemaphoreType.DMA,
            pltpu.SemaphoreType.DMA,
        ],
    )
    out = run(xyz_t, cen_t, features)
    return out.reshape(B, 3 + C, S, NSAMPLE)


# speedup vs baseline: 1.1702x; 1.1702x over previous
import jax
import jax.numpy as jnp
from jax import lax
from jax.experimental import pallas as pl
from jax.experimental.pallas import tpu as pltpu
from jax.experimental.pallas import tpu_sc as plsc

RADIUS = 0.2
NSAMPLE = 32

B = 2
N = 8192
S = 1024
C = 64

NUM_TILES = 16
CPT = S // NUM_TILES
LANES = 16
STEP = 2 * LANES
NSTEP = N // STEP
CH_PER_TILE = C // NUM_TILES
SCHUNK = 128
NSCHUNK = S // SCHUNK
BUFSZ = 64


def _body(xyz_hbm, cen_hbm, feat_hbm, out_hbm,
          pts_v, cen_v, buf_v, idxstage_v, idx_sh, idx_v, feat_v, stage_v,
          feat_sem):
    b = lax.axis_index("c")
    t = lax.axis_index("s")
    r2 = RADIUS * RADIUS

    feat_copies = []
    for q in range(CH_PER_TILE):
        ch = t * CH_PER_TILE + q
        feat_copies.append(pltpu.async_copy(
            feat_hbm.at[b, ch], feat_v.at[pl.ds(q * N, N)], feat_sem))

    pltpu.sync_copy(xyz_hbm.at[b], pts_v)
    pltpu.sync_copy(cen_hbm.at[b], cen_v)

    lane = lax.iota(jnp.int32, LANES)

    def center_body(ci, _):
        s = t * CPT + ci
        sv = jnp.full((LANES,), s, jnp.int32)
        cx = plsc.load_gather(cen_v, [sv])
        cy = plsc.load_gather(cen_v, [sv + S])
        cz = plsc.load_gather(cen_v, [sv + 2 * S])

        def dist_mask(base):
            xs = pts_v[pl.ds(base, LANES)]
            ys = pts_v[pl.ds(base + N, LANES)]
            zs = pts_v[pl.ds(base + 2 * N, LANES)]
            dx = xs - cx
            dy = ys - cy
            dz = zs - cz
            return dx * dx + dy * dy + dz * dz <= r2

        def cond(carry):
            i, count = carry
            return jnp.logical_and(i < NSTEP, count < NSAMPLE)

        def body(carry):
            i, count = carry
            base = pl.multiple_of(i * STEP, STEP)
            m0 = dist_mask(base)
            m1 = dist_mask(base + LANES)
            c0 = plsc.all_reduce_population_count(m0)[0]
            c1 = plsc.all_reduce_population_count(m1)[0]
            plsc.store_compressed(buf_v.at[pl.ds(count, LANES)],
                                  lane + base, mask=m0)
            plsc.store_compressed(buf_v.at[pl.ds(count + c0, LANES)],
                                  lane + (base + LANES), mask=m1)
            return i + 1, count + c0 + c1

        _, count = lax.while_loop(cond, body, (jnp.int32(0), jnp.int32(0)))

        first = plsc.load_gather(buf_v, [jnp.zeros((LANES,), jnp.int32)])
        fill = jnp.where(
            jnp.full((LANES,), count) == 0,
            jnp.full((LANES,), N - 1, jnp.int32), first)
        for j in range(NSAMPLE // LANES):
            pos = lane + j * LANES
            cur = buf_v[pl.ds(j * LANES, LANES)]
            res = jnp.where(pos < jnp.full((LANES,), count), cur, fill)
            idxstage_v[pl.ds(ci * NSAMPLE + j * LANES, LANES)] = res
        return 0

    lax.fori_loop(0, CPT, center_body, 0)

    pltpu.sync_copy(idxstage_v, idx_sh.at[pl.ds(t * CPT * NSAMPLE,
                                                CPT * NSAMPLE)])
    plsc.subcore_barrier()
    pltpu.sync_copy(idx_sh, idx_v)

    def xyz_body(ci, _):
        s = t * CPT + ci
        sv = jnp.full((LANES,), s, jnp.int32)
        cens = [plsc.load_gather(cen_v, [sv + d * S]) for d in range(3)]
        for j in range(NSAMPLE // LANES):
            idxv = idxstage_v[pl.ds(ci * NSAMPLE + j * LANES, LANES)]
            for d in range(3):
                vals = plsc.load_gather(pts_v, [idxv + d * N]) - cens[d]
                stage_v[pl.ds(d * CPT * NSAMPLE + ci * NSAMPLE + j * LANES,
                              LANES)] = vals
        return 0

    lax.fori_loop(0, CPT, xyz_body, 0)
    for d in range(3):
        pltpu.sync_copy(
            stage_v.at[pl.ds(d * CPT * NSAMPLE, CPT * NSAMPLE)],
            out_hbm.at[b, d, pl.ds(t * CPT * NSAMPLE, CPT * NSAMPLE)])

    for cp in feat_copies:
        cp.wait()

    def chunk_body(k, _):
        def cbody(ci, _):
            s = k * SCHUNK + ci
            for j in range(NSAMPLE // LANES):
                idxv = idx_v[pl.ds(s * NSAMPLE + j * LANES, LANES)]
                for q in range(CH_PER_TILE):
                    vals = plsc.load_gather(feat_v, [idxv + q * N])
                    stage_v[pl.ds(q * SCHUNK * NSAMPLE + ci * NSAMPLE
                                  + j * LANES, LANES)] = vals
            return 0
        lax.fori_loop(0, SCHUNK, cbody, 0)
        for q in range(CH_PER_TILE):
            ch = t * CH_PER_TILE + q
            pltpu.sync_copy(
                stage_v.at[pl.ds(q * SCHUNK * NSAMPLE, SCHUNK * NSAMPLE)],
                out_hbm.at[b, 3 + ch, pl.ds(k * SCHUNK * NSAMPLE,
                                            SCHUNK * NSAMPLE)])
        return 0

    lax.fori_loop(0, NSCHUNK, chunk_body, 0)


@jax.jit
def kernel(xyz, center_xyz, features):
    xyz_t = jnp.transpose(xyz, (0, 2, 1)).reshape(B, 3 * N)
    cen_t = jnp.transpose(center_xyz, (0, 2, 1)).reshape(B, 3 * S)

    mesh = plsc.VectorSubcoreMesh(core_axis_name="c", subcore_axis_name="s",
                                  num_cores=2, num_subcores=NUM_TILES)
    run = pl.kernel(
        _body,
        out_type=jax.ShapeDtypeStruct((B, 3 + C, S * NSAMPLE), jnp.float32),
        mesh=mesh,
        compiler_params=pltpu.CompilerParams(needs_layout_passes=False),
        scratch_types=[
            pltpu.VMEM((3 * N,), jnp.float32),
            pltpu.VMEM((3 * S,), jnp.float32),
            pltpu.VMEM((BUFSZ,), jnp.int32),
            pltpu.VMEM((CPT * NSAMPLE,), jnp.int32),
            pltpu.VMEM_SHARED((S * NSAMPLE,), jnp.int32),
            pltpu.VMEM((S * NSAMPLE,), jnp.int32),
            pltpu.VMEM((CH_PER_TILE * N,), jnp.float32),
            pltpu.VMEM((CH_PER_TILE * SCHUNK * NSAMPLE,),
                       jnp.float32),
            pltpu.SemaphoreType.DMA,
        ],
    )
    out = run(xyz_t, cen_t, features)
    return out.reshape(B, 3 + C, S, NSAMPLE)


# speedup vs baseline: 1.7754x; 1.5172x over previous
import jax
import jax.numpy as jnp
from jax import lax
from jax.experimental import pallas as pl
from jax.experimental.pallas import tpu as pltpu
from jax.experimental.pallas import tpu_sc as plsc

RADIUS = 0.2
NSAMPLE = 32

B = 2
N = 8192
S = 1024
C = 64

NUM_TILES = 16
CPT = S // NUM_TILES
LANES = 16
KCHUNK = 4
STEP = KCHUNK * LANES
NSTEP = N // STEP
CH_PER_TILE = C // NUM_TILES
SCHUNK = 128
NSCHUNK = S // SCHUNK
BUFSZ = NSAMPLE + STEP + LANES


def _body(xyz_hbm, cen_hbm, feat_hbm, out_hbm,
          pts_v, cen_v, buf_v, idxstage_v, idx_sh, idx_v, feat_v, stage_v,
          feat_sem):
    b = lax.axis_index("c")
    t = lax.axis_index("s")
    r2 = RADIUS * RADIUS

    feat_copies = []
    for q in range(CH_PER_TILE):
        ch = t * CH_PER_TILE + q
        feat_copies.append(pltpu.async_copy(
            feat_hbm.at[b, ch], feat_v.at[pl.ds(q * N, N)], feat_sem))

    pltpu.sync_copy(xyz_hbm.at[b], pts_v)
    pltpu.sync_copy(cen_hbm.at[b], cen_v)

    lane = lax.iota(jnp.int32, LANES)

    def center_body(ci, _):
        s = t * CPT + ci
        sv = jnp.full((LANES,), s, jnp.int32)
        cx = plsc.load_gather(cen_v, [sv])
        cy = plsc.load_gather(cen_v, [sv + S])
        cz = plsc.load_gather(cen_v, [sv + 2 * S])

        def dist_mask(base):
            xs = pts_v[pl.ds(base, LANES)]
            ys = pts_v[pl.ds(base + N, LANES)]
            zs = pts_v[pl.ds(base + 2 * N, LANES)]
            dx = xs - cx
            dy = ys - cy
            dz = zs - cz
            return dx * dx + dy * dy + dz * dz <= r2

        def cond(carry):
            i, count = carry
            return jnp.logical_and(i < NSTEP, count < NSAMPLE)

        def body(carry):
            i, count = carry
            base = pl.multiple_of(i * STEP, STEP)
            ms = [dist_mask(base + c * LANES) for c in range(KCHUNK)]
            cs = [plsc.all_reduce_population_count(m)[0] for m in ms]
            off = count
            for c in range(KCHUNK):
                plsc.store_compressed(buf_v.at[pl.ds(off, LANES)],
                                      lane + (base + c * LANES), mask=ms[c])
                off = off + cs[c]
            return i + 1, off

        _, count = lax.while_loop(cond, body, (jnp.int32(0), jnp.int32(0)))

        first = plsc.load_gather(buf_v, [jnp.zeros((LANES,), jnp.int32)])
        fill = jnp.where(
            jnp.full((LANES,), count) == 0,
            jnp.full((LANES,), N - 1, jnp.int32), first)
        for j in range(NSAMPLE // LANES):
            pos = lane + j * LANES
            cur = buf_v[pl.ds(j * LANES, LANES)]
            res = jnp.where(pos < jnp.full((LANES,), count), cur, fill)
            idxstage_v[pl.ds(ci * NSAMPLE + j * LANES, LANES)] = res
        return 0

    lax.fori_loop(0, CPT, center_body, 0)

    pltpu.sync_copy(idxstage_v, idx_sh.at[pl.ds(t * CPT * NSAMPLE,
                                                CPT * NSAMPLE)])
    plsc.subcore_barrier()
    pltpu.sync_copy(idx_sh, idx_v)

    @plsc.parallel_loop(0, CPT, unroll=2)
    def xyz_body(ci):
        s = t * CPT + ci
        sv = jnp.full((LANES,), s, jnp.int32)
        cens = [plsc.load_gather(cen_v, [sv + d * S]) for d in range(3)]
        for j in range(NSAMPLE // LANES):
            idxv = idxstage_v[pl.ds(ci * NSAMPLE + j * LANES, LANES)]
            for d in range(3):
                vals = plsc.load_gather(pts_v, [idxv + d * N]) - cens[d]
                stage_v[pl.ds(d * CPT * NSAMPLE + ci * NSAMPLE + j * LANES,
                              LANES)] = vals
    for d in range(3):
        pltpu.sync_copy(
            stage_v.at[pl.ds(d * CPT * NSAMPLE, CPT * NSAMPLE)],
            out_hbm.at[b, d, pl.ds(t * CPT * NSAMPLE, CPT * NSAMPLE)])

    for cp in feat_copies:
        cp.wait()

    def chunk_body(k, _):
        @plsc.parallel_loop(0, SCHUNK, unroll=2)
        def cbody(ci):
            s = k * SCHUNK + ci
            for j in range(NSAMPLE // LANES):
                idxv = idx_v[pl.ds(s * NSAMPLE + j * LANES, LANES)]
                for q in range(CH_PER_TILE):
                    vals = plsc.load_gather(feat_v, [idxv + q * N])
                    stage_v[pl.ds(q * SCHUNK * NSAMPLE + ci * NSAMPLE
                                  + j * LANES, LANES)] = vals
        for q in range(CH_PER_TILE):
            ch = t * CH_PER_TILE + q
            pltpu.sync_copy(
                stage_v.at[pl.ds(q * SCHUNK * NSAMPLE, SCHUNK * NSAMPLE)],
                out_hbm.at[b, 3 + ch, pl.ds(k * SCHUNK * NSAMPLE,
                                            SCHUNK * NSAMPLE)])
        return 0

    lax.fori_loop(0, NSCHUNK, chunk_body, 0)


@jax.jit
def kernel(xyz, center_xyz, features):
    xyz_t = jnp.transpose(xyz, (0, 2, 1)).reshape(B, 3 * N)
    cen_t = jnp.transpose(center_xyz, (0, 2, 1)).reshape(B, 3 * S)

    mesh = plsc.VectorSubcoreMesh(core_axis_name="c", subcore_axis_name="s",
                                  num_cores=2, num_subcores=NUM_TILES)
    run = pl.kernel(
        _body,
        out_type=jax.ShapeDtypeStruct((B, 3 + C, S * NSAMPLE), jnp.float32),
        mesh=mesh,
        compiler_params=pltpu.CompilerParams(needs_layout_passes=False),
        scratch_types=[
            pltpu.VMEM((3 * N,), jnp.float32),
            pltpu.VMEM((3 * S,), jnp.float32),
            pltpu.VMEM((BUFSZ,), jnp.int32),
            pltpu.VMEM((CPT * NSAMPLE,), jnp.int32),
            pltpu.VMEM_SHARED((S * NSAMPLE,), jnp.int32),
            pltpu.VMEM((S * NSAMPLE,), jnp.int32),
            pltpu.VMEM((CH_PER_TILE * N,), jnp.float32),
            pltpu.VMEM((CH_PER_TILE * SCHUNK * NSAMPLE,),
                       jnp.float32),
            pltpu.SemaphoreType.DMA,
        ],
    )
    out = run(xyz_t, cen_t, features)
    return out.reshape(B, 3 + C, S, NSAMPLE)


# speedup vs baseline: 1.9694x; 1.1093x over previous
import jax
import jax.numpy as jnp
from jax import lax
from jax.experimental import pallas as pl
from jax.experimental.pallas import tpu as pltpu
from jax.experimental.pallas import tpu_sc as plsc

RADIUS = 0.2
NSAMPLE = 32

B = 2
N = 8192
S = 1024
C = 64

NUM_TILES = 16
CPT = S // NUM_TILES
LANES = 16
KCHUNK = 8
STEP = KCHUNK * LANES
NSTEP = N // STEP
CH_PER_TILE = C // NUM_TILES
SCHUNK = 128
NSCHUNK = S // SCHUNK
BUFSZ = NSAMPLE + STEP + LANES


def _body(xyz_hbm, cen_hbm, feat_hbm, out_hbm,
          pts_v, cen_v, buf_v, idxstage_v, idx_sh, idx_v, feat_v, stage_v,
          feat_sem):
    b = lax.axis_index("c")
    t = lax.axis_index("s")
    r2 = RADIUS * RADIUS

    feat_copies = []
    for q in range(CH_PER_TILE):
        ch = t * CH_PER_TILE + q
        feat_copies.append(pltpu.async_copy(
            feat_hbm.at[b, ch], feat_v.at[pl.ds(q * N, N)], feat_sem))

    pltpu.sync_copy(xyz_hbm.at[b], pts_v)
    pltpu.sync_copy(cen_hbm.at[b], cen_v)

    lane = lax.iota(jnp.int32, LANES)

    def center_body(ci, _):
        s = t * CPT + ci
        sv = jnp.full((LANES,), s, jnp.int32)
        cx = plsc.load_gather(cen_v, [sv])
        cy = plsc.load_gather(cen_v, [sv + S])
        cz = plsc.load_gather(cen_v, [sv + 2 * S])

        def dist_mask(base):
            xs = pts_v[pl.ds(base, LANES)]
            ys = pts_v[pl.ds(base + N, LANES)]
            zs = pts_v[pl.ds(base + 2 * N, LANES)]
            dx = xs - cx
            dy = ys - cy
            dz = zs - cz
            return dx * dx + dy * dy + dz * dz <= r2

        def cond(carry):
            i, count = carry
            return jnp.logical_and(i < NSTEP, count < NSAMPLE)

        def body(carry):
            i, count = carry
            base = pl.multiple_of(i * STEP, STEP)
            ms = [dist_mask(base + c * LANES) for c in range(KCHUNK)]
            cs = [plsc.all_reduce_population_count(m)[0] for m in ms]
            off = count
            for c in range(KCHUNK):
                plsc.store_compressed(buf_v.at[pl.ds(off, LANES)],
                                      lane + (base + c * LANES), mask=ms[c])
                off = off + cs[c]
            return i + 1, off

        _, count = lax.while_loop(cond, body, (jnp.int32(0), jnp.int32(0)))

        first = plsc.load_gather(buf_v, [jnp.zeros((LANES,), jnp.int32)])
        fill = jnp.where(
            jnp.full((LANES,), count) == 0,
            jnp.full((LANES,), N - 1, jnp.int32), first)
        for j in range(NSAMPLE // LANES):
            pos = lane + j * LANES
            cur = buf_v[pl.ds(j * LANES, LANES)]
            res = jnp.where(pos < jnp.full((LANES,), count), cur, fill)
            idxstage_v[pl.ds(ci * NSAMPLE + j * LANES, LANES)] = res
        return 0

    lax.fori_loop(0, CPT, center_body, 0)

    pltpu.sync_copy(idxstage_v, idx_sh.at[pl.ds(t * CPT * NSAMPLE,
                                                CPT * NSAMPLE)])
    plsc.subcore_barrier()
    pltpu.sync_copy(idx_sh, idx_v)

    @plsc.parallel_loop(0, CPT, unroll=4)
    def xyz_body(ci):
        s = t * CPT + ci
        sv = jnp.full((LANES,), s, jnp.int32)
        cens = [plsc.load_gather(cen_v, [sv + d * S]) for d in range(3)]
        for j in range(NSAMPLE // LANES):
            idxv = idxstage_v[pl.ds(ci * NSAMPLE + j * LANES, LANES)]
            for d in range(3):
                vals = plsc.load_gather(pts_v, [idxv + d * N]) - cens[d]
                stage_v[pl.ds(d * CPT * NSAMPLE + ci * NSAMPLE + j * LANES,
                              LANES)] = vals
    for d in range(3):
        pltpu.sync_copy(
            stage_v.at[pl.ds(d * CPT * NSAMPLE, CPT * NSAMPLE)],
            out_hbm.at[b, d, pl.ds(t * CPT * NSAMPLE, CPT * NSAMPLE)])

    for cp in feat_copies:
        cp.wait()

    def chunk_body(k, _):
        @plsc.parallel_loop(0, SCHUNK, unroll=4)
        def cbody(ci):
            s = k * SCHUNK + ci
            for j in range(NSAMPLE // LANES):
                idxv = idx_v[pl.ds(s * NSAMPLE + j * LANES, LANES)]
                for q in range(CH_PER_TILE):
                    vals = plsc.load_gather(feat_v, [idxv + q * N])
                    stage_v[pl.ds(q * SCHUNK * NSAMPLE + ci * NSAMPLE
                                  + j * LANES, LANES)] = vals
        for q in range(CH_PER_TILE):
            ch = t * CH_PER_TILE + q
            pltpu.sync_copy(
                stage_v.at[pl.ds(q * SCHUNK * NSAMPLE, SCHUNK * NSAMPLE)],
                out_hbm.at[b, 3 + ch, pl.ds(k * SCHUNK * NSAMPLE,
                                            SCHUNK * NSAMPLE)])
        return 0

    lax.fori_loop(0, NSCHUNK, chunk_body, 0)


@jax.jit
def kernel(xyz, center_xyz, features):
    xyz_t = jnp.transpose(xyz, (0, 2, 1)).reshape(B, 3 * N)
    cen_t = jnp.transpose(center_xyz, (0, 2, 1)).reshape(B, 3 * S)

    mesh = plsc.VectorSubcoreMesh(core_axis_name="c", subcore_axis_name="s",
                                  num_cores=2, num_subcores=NUM_TILES)
    run = pl.kernel(
        _body,
        out_type=jax.ShapeDtypeStruct((B, 3 + C, S * NSAMPLE), jnp.float32),
        mesh=mesh,
        compiler_params=pltpu.CompilerParams(needs_layout_passes=False),
        scratch_types=[
            pltpu.VMEM((3 * N,), jnp.float32),
            pltpu.VMEM((3 * S,), jnp.float32),
            pltpu.VMEM((BUFSZ,), jnp.int32),
            pltpu.VMEM((CPT * NSAMPLE,), jnp.int32),
            pltpu.VMEM_SHARED((S * NSAMPLE,), jnp.int32),
            pltpu.VMEM((S * NSAMPLE,), jnp.int32),
            pltpu.VMEM((CH_PER_TILE * N,), jnp.float32),
            pltpu.VMEM((CH_PER_TILE * SCHUNK * NSAMPLE,),
                       jnp.float32),
            pltpu.SemaphoreType.DMA,
        ],
    )
    out = run(xyz_t, cen_t, features)
    return out.reshape(B, 3 + C, S, NSAMPLE)
